# baseline (speedup 1.0000x reference)
import functools

import jax
import jax.numpy as jnp
from jax import lax
from jax.experimental import pallas as pl
from jax.experimental.pallas import tpu as pltpu
from jax.experimental.pallas import tpu_sc as plsc

NUM_EMB = 1000000
D = 64
BATCH = 16384
HIST = 20
B = BATCH * HIST

NC, NS = 2, 16
NW = NC * NS
PER_W = B // NW
RC = 16
C = RC * HIST
CH = PER_W // C
ROWS_W = BATCH // NW
NB = 4


def _make_gather():
  mesh = plsc.VectorSubcoreMesh(core_axis_name="c", subcore_axis_name="s")

  @functools.partial(
      pl.kernel,
      out_type=jax.ShapeDtypeStruct((BATCH, HIST, D), jnp.float32),
      mesh=mesh,
      scratch_types=[
          pltpu.VMEM((PER_W,), jnp.int32),
          pltpu.VMEM((NB, C, D), jnp.float32),
          pltpu.SemaphoreType.DMA((NB,)),
          pltpu.SemaphoreType.DMA((NB,)),
      ],
      compiler_params=pltpu.CompilerParams(use_tc_tiling_on_sc=False),
  )
  def gather_kernel(idx_hbm, table_hbm, out_hbm, idx_v, bufs, gsem, osem):
    wid = lax.axis_index("s") * NC + lax.axis_index("c")
    base_r = wid * ROWS_W

    pltpu.sync_copy(idx_hbm.at[pl.ds(wid * PER_W, PER_W)], idx_v)

    def gather(j, b):
      return pltpu.make_async_copy(
          table_hbm.at[idx_v.at[pl.ds(j * C, C)]], bufs.at[b], gsem.at[b]
      )

    def out_copy(j, b, r):
      return pltpu.make_async_copy(
          bufs.at[b, pl.ds(r * HIST, HIST)],
          out_hbm.at[base_r + j * RC + r],
          osem.at[b],
      )

    for b in range(NB):
      gather(b, b).start()

    @pl.loop(0, CH - NB, step=NB)
    def _main(j0):
      for b in range(NB):
        j = j0 + b
        gather(j, b).wait()
        for r in range(RC):
          out_copy(j, b, r).start()
        for r in range(RC):
          out_copy(j, b, r).wait()
        gather(j + NB, b).start()

    for b in range(NB):
      j = CH - NB + b
      gather(j, b).wait()
      for r in range(RC):
        out_copy(j, b, r).start()
      for r in range(RC):
        out_copy(j, b, r).wait()

  return gather_kernel


_gather = _make_gather()


@jax.jit
def kernel(input, table):
  return _gather(input.reshape(B), table)


# speedup vs baseline: 1.0015x; 1.0015x over previous
import functools

import jax
import jax.numpy as jnp
from jax import lax
from jax.experimental import pallas as pl
from jax.experimental.pallas import tpu as pltpu
from jax.experimental.pallas import tpu_sc as plsc

NUM_EMB = 1000000
D = 64
BATCH = 16384
HIST = 20
B = BATCH * HIST

NC, NS = 2, 16
NW = NC * NS
PER_W = B // NW

IW = 128
IR_W = PER_W // IW
IR_C = 5
C = IR_C * IW
CH = IR_W // IR_C
RC = C // HIST
ROWS_W = BATCH // NW
NB = 2


def _make_gather():
  mesh = plsc.VectorSubcoreMesh(core_axis_name="c", subcore_axis_name="s")

  @functools.partial(
      pl.kernel,
      out_type=jax.ShapeDtypeStruct((BATCH, HIST, D), jnp.float32),
      mesh=mesh,
      scratch_types=[
          pltpu.VMEM((IR_W, IW), jnp.int32),
          pltpu.VMEM((NB, C, D), jnp.float32),
          pltpu.SemaphoreType.DMA((NB,)),
          pltpu.SemaphoreType.DMA((NB,)),
      ],
      compiler_params=pltpu.CompilerParams(use_tc_tiling_on_sc=False),
  )
  def gather_kernel(idx_hbm, table_hbm, out_hbm, idx_v, bufs, gsem, osem):
    wid = lax.axis_index("s") * NC + lax.axis_index("c")
    base_r = wid * ROWS_W

    pltpu.sync_copy(idx_hbm.at[pl.ds(wid * IR_W, IR_W)], idx_v)

    def gathers(j, b):
      return [
          pltpu.make_async_copy(
              table_hbm.at[idx_v.at[j * IR_C + k]],
              bufs.at[b, pl.ds(k * IW, IW)],
              gsem.at[b],
          )
          for k in range(IR_C)
      ]

    def out_copy(j, b, r):
      return pltpu.make_async_copy(
          bufs.at[b, pl.ds(r * HIST, HIST)],
          out_hbm.at[base_r + j * RC + r],
          osem.at[b],
      )

    for b in range(NB):
      for g in gathers(b, b):
        g.start()

    @pl.loop(0, CH - NB, step=NB)
    def _main(j0):
      for b in range(NB):
        j = j0 + b
        for g in gathers(j, b):
          g.wait()
        for r in range(RC):
          out_copy(j, b, r).start()
        for r in range(RC):
          out_copy(j, b, r).wait()
        for g in gathers(j + NB, b):
          g.start()

    for b in range(NB):
      j = CH - NB + b
      for g in gathers(j, b):
        g.wait()
      for r in range(RC):
        out_copy(j, b, r).start()
      for r in range(RC):
        out_copy(j, b, r).wait()

  return gather_kernel


_gather = _make_gather()


@jax.jit
def kernel(input, table):
  return _gather(input.reshape(B // IW, IW), table)
